# baseline (speedup 1.0000x reference)
import functools
import math

import jax
import jax.numpy as jnp
from jax import lax
from jax.experimental import pallas as pl
from jax.experimental.pallas import tpu as pltpu
from jax.experimental.pallas import tpu_sc as plsc

D_MODEL = 64
VOCAB = 1000000
SCALE = math.sqrt(D_MODEL)

_info = plsc.get_sparse_core_info()
NC, NS, L = _info.num_cores, _info.num_subcores, _info.num_lanes
NW = NC * NS

B_TOTAL = 4096 * 200
B_PER_W = B_TOTAL // NW
CHUNK = 128
N_CHUNKS = B_PER_W // CHUNK


@functools.partial(
    pl.kernel,
    out_type=jax.ShapeDtypeStruct((B_TOTAL, D_MODEL), jnp.float32),
    mesh=plsc.VectorSubcoreMesh(core_axis_name="c", subcore_axis_name="s"),
    scratch_types=[
        pltpu.VMEM((CHUNK,), jnp.int32),
        pltpu.VMEM((CHUNK, D_MODEL), jnp.float32),
        pltpu.SemaphoreType.DMA,
    ],
    compiler_params=pltpu.CompilerParams(use_tc_tiling_on_sc=False),
)
def _embed_sc(lut_hbm, idx_hbm, out_hbm, idx_v, rows_v, sem):
    wid = lax.axis_index("s") * NC + lax.axis_index("c")
    wbase = wid * B_PER_W

    def chunk_body(i, carry):
        base = pl.multiple_of(wbase + i * CHUNK, CHUNK)
        pltpu.sync_copy(idx_hbm.at[pl.ds(base, CHUNK)], idx_v)
        pltpu.async_copy(lut_hbm.at[idx_v], rows_v, sem).wait()

        def scale_row(r, c2):
            for j in range(D_MODEL // 16):
                sl = (r, pl.ds(j * 16, 16))
                rows_v[sl] = rows_v[sl] * SCALE
            return c2

        lax.fori_loop(0, CHUNK, scale_row, 0)
        pltpu.sync_copy(rows_v, out_hbm.at[pl.ds(base, CHUNK)])
        return carry

    lax.fori_loop(0, N_CHUNKS, chunk_body, 0)


def kernel(x, lut):
    xf = x.reshape(-1).astype(jnp.int32)
    out = _embed_sc(lut, xf)
    return out.reshape(x.shape + (D_MODEL,))


# speedup vs baseline: 1.2226x; 1.2226x over previous
import functools
import math

import jax
import jax.numpy as jnp
from jax import lax
from jax.experimental import pallas as pl
from jax.experimental.pallas import tpu as pltpu
from jax.experimental.pallas import tpu_sc as plsc

D_MODEL = 64
SCALE = math.sqrt(D_MODEL)

_info = plsc.get_sparse_core_info()
NC, NS, L = _info.num_cores, _info.num_subcores, _info.num_lanes
NW = NC * NS

CHUNK = 128
B_TOTAL = 4096 * 200
N_ROWS = B_TOTAL // CHUNK
ROWS_PER_W = N_ROWS // NW
NBUF = 8
N_BLOCKS = ROWS_PER_W // NBUF


@functools.partial(
    pl.kernel,
    out_type=jax.ShapeDtypeStruct((B_TOTAL, D_MODEL), jnp.float32),
    mesh=plsc.VectorSubcoreMesh(core_axis_name="c", subcore_axis_name="s"),
    scratch_types=[
        pltpu.VMEM((ROWS_PER_W, CHUNK), jnp.int32),
        pltpu.VMEM((NBUF, CHUNK, D_MODEL), jnp.float32),
        pltpu.SemaphoreType.DMA((NBUF,)),
        pltpu.SemaphoreType.DMA((NBUF,)),
    ],
    compiler_params=pltpu.CompilerParams(use_tc_tiling_on_sc=False),
)
def _embed_sc(lut_hbm, idx_hbm, out_hbm, idx_v, rows_v, gsem, osem):
    wid = lax.axis_index("s") * NC + lax.axis_index("c")
    wrow0 = wid * ROWS_PER_W

    pltpu.sync_copy(idx_hbm.at[pl.ds(wrow0, ROWS_PER_W)], idx_v)

    def block_body(t, carry):
        chunk0 = t * NBUF
        for b in range(NBUF):
            pltpu.async_copy(
                lut_hbm.at[idx_v.at[chunk0 + b]], rows_v.at[b], gsem.at[b]
            )
        for b in range(NBUF):
            pltpu.make_async_copy(
                lut_hbm.at[idx_v.at[chunk0 + b]], rows_v.at[b], gsem.at[b]
            ).wait()

            def scale_row(r, c2):
                for j in range(D_MODEL // L):
                    rows_v[b, r, pl.ds(j * L, L)] = (
                        rows_v[b, r, pl.ds(j * L, L)] * SCALE
                    )
                return c2

            lax.fori_loop(0, CHUNK, scale_row, 0)
            base = pl.multiple_of((wrow0 + chunk0 + b) * CHUNK, CHUNK)
            pltpu.async_copy(
                rows_v.at[b], out_hbm.at[pl.ds(base, CHUNK)], osem.at[b]
            )
        for b in range(NBUF):
            pltpu.make_async_copy(
                rows_v.at[b], out_hbm.at[pl.ds(0, CHUNK)], osem.at[b]
            ).wait()
        return carry

    lax.fori_loop(0, N_BLOCKS, block_body, 0)


def kernel(x, lut):
    xf = x.reshape(N_ROWS, CHUNK).astype(jnp.int32)
    out = _embed_sc(lut, xf)
    return out.reshape(x.shape + (D_MODEL,))
